# baseline (speedup 1.0000x reference)
import functools
import math

import jax
import jax.numpy as jnp
from jax import lax
from jax.experimental import pallas as pl
from jax.experimental.pallas import tpu as pltpu
from jax.experimental.pallas import tpu_sc as plsc

D = 512
SQRT_D = math.sqrt(D)
NC = 2
NS = 16
NW = NC * NS
BB = 8
SLICES = (256, 256, 256, 256)


def _sc_gather_add(tok_table, pos_table, tok_ids, pos_ids, CH):
    n = tok_ids.shape[0]
    per_w = n // NW
    NBUF = 4
    assert per_w * NW == n and per_w % (NBUF * CH) == 0

    mesh = plsc.VectorSubcoreMesh(core_axis_name="c", subcore_axis_name="s")

    rows_t = pltpu.VMEM((CH, D), jnp.float32)
    @functools.partial(
        pl.kernel,
        out_type=jax.ShapeDtypeStruct((n, D), jnp.float32),
        mesh=mesh,
        scratch_types=[
            pltpu.VMEM((per_w,), jnp.int32),
            pltpu.VMEM((per_w,), jnp.int32),
        ] + [rows_t] * (2 * NBUF)
          + [pltpu.SemaphoreType.DMA] * (3 * NBUF),
    )
    def sc_kernel(tok_tbl, pos_tbl, tok_idx, pos_idx, out, ti_all, pi_all,
                  *bufsem):
        trs = bufsem[0:NBUF]
        prs = bufsem[NBUF:2 * NBUF]
        gts = bufsem[2 * NBUF:3 * NBUF]
        gps = bufsem[3 * NBUF:4 * NBUF]
        wss = bufsem[4 * NBUF:5 * NBUF]
        wid = lax.axis_index("s") * NC + lax.axis_index("c")
        base = wid * per_w
        pltpu.sync_copy(tok_idx.at[pl.ds(base, per_w)], ti_all)
        pltpu.sync_copy(pos_idx.at[pl.ds(base, per_w)], pi_all)

        def start_gathers(off, b):
            pltpu.async_copy(tok_tbl.at[ti_all.at[pl.ds(off, CH)]], trs[b],
                             gts[b])
            pltpu.async_copy(pos_tbl.at[pi_all.at[pl.ds(off, CH)]], prs[b],
                             gps[b])

        start_gathers(0, 0)
        start_gathers(CH, 1)

        @pl.loop(0, per_w, step=NBUF * CH)
        def _(c):
            for b in range(NBUF):
                tr, pr = trs[b], prs[b]
                off = c + b * CH
                tb = (b + 2) % NBUF
                nxt = off + 2 * CH

                @pl.when(nxt < per_w)
                def _():
                    @pl.when(off >= 2 * CH)
                    def _():
                        pltpu.make_async_copy(
                            trs[tb],
                            out.at[pl.ds(base + off - 2 * CH, CH)],
                            wss[tb]).wait()
                    start_gathers(nxt, tb)

                pltpu.make_async_copy(
                    tok_tbl.at[ti_all.at[pl.ds(off, CH)]], tr, gts[b]).wait()
                pltpu.make_async_copy(
                    pos_tbl.at[pi_all.at[pl.ds(off, CH)]], pr, gps[b]).wait()

                @pl.loop(0, CH)
                def _(r):
                    trow = tr.at[r]
                    prow = pr.at[r]
                    for cc in range(D // 16):
                        slc = pl.ds(cc * 16, 16)
                        trow[slc] = trow[slc] * SQRT_D + prow[slc]

                pltpu.async_copy(trs[b], out.at[pl.ds(base + off, CH)],
                                 wss[b])

        for b in range(NBUF):
            pltpu.make_async_copy(
                trs[b], out.at[pl.ds(base + per_w - (NBUF - b) * CH, CH)],
                wss[b]).wait()

    return sc_kernel(tok_table, pos_table, tok_ids, pos_ids)


def _tc_body(*refs):
    (part_ref, syn_ref, et_ref, etab_ref, pe_ref, w1_ref, b1_ref, g1_ref,
     bb1_ref, w2_ref, g_ref, bb_ref, out_ref) = refs[-13:]
    s_len = part_ref.shape[1]
    x = part_ref[...]
    etf = et_ref[0].astype(jnp.float32)
    etb = lax.broadcast_in_dim(etf, (BB, s_len, D), (0, 1))
    e0 = etab_ref[0:1, :][None]
    e1 = etab_ref[1:2, :][None]
    e2 = etab_ref[2:3, :][None]
    elem = jnp.where(etb == 0.0, e0, jnp.where(etb == 1.0, e1, e2))

    s2 = syn_ref[...].reshape(BB * s_len, 3)
    h = jnp.dot(s2, w1_ref[...], preferred_element_type=jnp.float32)
    h = h + b1_ref[...]
    m1 = jnp.mean(h, axis=-1, keepdims=True)
    v1 = jnp.mean(h * h, axis=-1, keepdims=True) - m1 * m1
    h = (h - m1) * lax.rsqrt(v1 + 1e-5) * g1_ref[...] + bb1_ref[...]
    h = jnp.maximum(h, 0.0)
    se = jnp.dot(h.astype(jnp.bfloat16), w2_ref[...],
                 preferred_element_type=jnp.float32)
    se = se.reshape(BB, s_len, D)

    comb = x + elem + se + pe_ref[...][None]
    m2 = jnp.mean(comb, axis=-1, keepdims=True)
    v2 = jnp.mean(comb * comb, axis=-1, keepdims=True) - m2 * m2
    out_ref[...] = (comb - m2) * lax.rsqrt(v2 + 1e-5) * g_ref[...] + bb_ref[...]


def _tc_combine_slice(prev, partial, syntax, et3, etab8, pe_s, w1, b1, ln1_g,
                      ln1_b, w2, ln_g, ln_b, blk_off, out_sds):
    sb, s_len, _ = partial.shape
    nb = sb // BB
    grid = (nb,)
    prev_specs = [] if prev is None else [pl.BlockSpec(memory_space=pl.ANY)]
    prev_args = () if prev is None else (prev,)
    aliases = {} if prev is None else {0: 0}
    return pl.pallas_call(
        _tc_body,
        grid=grid,
        in_specs=prev_specs + [
            pl.BlockSpec((BB, s_len, D), lambda i: (i, 0, 0)),
            pl.BlockSpec((BB, s_len, 3), lambda i: (i, 0, 0)),
            pl.BlockSpec((1, BB, s_len), lambda i: (i, 0, 0)),
            pl.BlockSpec((8, D), lambda i: (0, 0)),
            pl.BlockSpec((s_len, D), lambda i: (0, 0)),
            pl.BlockSpec((3, D // 4), lambda i: (0, 0)),
            pl.BlockSpec((1, D // 4), lambda i: (0, 0)),
            pl.BlockSpec((1, D // 4), lambda i: (0, 0)),
            pl.BlockSpec((1, D // 4), lambda i: (0, 0)),
            pl.BlockSpec((D // 4, D), lambda i: (0, 0)),
            pl.BlockSpec((1, D), lambda i: (0, 0)),
            pl.BlockSpec((1, D), lambda i: (0, 0)),
        ],
        out_specs=pl.BlockSpec((BB, s_len, D),
                               lambda i: (i + blk_off, 0, 0)),
        out_shape=out_sds,
        input_output_aliases=aliases,
    )(*prev_args, partial, syntax, et3, etab8, pe_s, w1, b1, ln1_g, ln1_b,
      w2, ln_g, ln_b)


def kernel(token_ids, pos_tags, elem_types, syntax_features, tok_table,
           pos_tag_table, elem_table, pe, w1, b1, ln1_g, ln1_b, w2, b2,
           ln_g, ln_b):
    b_sz, s_len = token_ids.shape
    assert sum(SLICES) == b_sz
    tok_flat = token_ids.reshape(-1).astype(jnp.int32)
    pos_flat = pos_tags.reshape(-1).astype(jnp.int32)
    et_all = elem_types.astype(jnp.int32)
    pe_s = pe[0, :s_len, :] + b2[None, :]
    etab8 = jnp.zeros((8, D), jnp.float32).at[:3].set(elem_table)
    w2b = w2.astype(jnp.bfloat16)
    b1r = b1.reshape(1, -1)
    g1r = ln1_g.reshape(1, -1)
    bb1r = ln1_b.reshape(1, -1)
    gr = ln_g.reshape(1, -1)
    bbr = ln_b.reshape(1, -1)
    out_sds = jax.ShapeDtypeStruct((b_sz, s_len, D), jnp.float32)

    starts = [sum(SLICES[:i]) for i in range(len(SLICES))]
    partials = []
    for st, sb in zip(starts, SLICES):
        n_sl = sb * s_len
        per_w = n_sl // NW
        ch = 16 if per_w % 64 == 0 else 8
        p = _sc_gather_add(tok_table, pos_tag_table,
                           lax.dynamic_slice(tok_flat, (st * s_len,), (n_sl,)),
                           lax.dynamic_slice(pos_flat, (st * s_len,), (n_sl,)),
                           ch)
        partials.append(p.reshape(sb, s_len, D))

    out = None
    for i, (st, sb) in enumerate(zip(starts, SLICES)):
        syn_i = lax.dynamic_slice(syntax_features, (st, 0, 0), (sb, s_len, 3))
        et_i = lax.dynamic_slice(et_all, (st, 0), (sb, s_len))
        et3_i = et_i.reshape(sb // BB, BB, s_len)
        out = _tc_combine_slice(out, partials[i], syn_i, et3_i, etab8, pe_s,
                                w1, b1r, g1r, bb1r, w2b, gr, bbr,
                                blk_off=st // BB, out_sds=out_sds)
    return out


# speedup vs baseline: 1.0137x; 1.0137x over previous
import functools
import math

import jax
import jax.numpy as jnp
from jax import lax
from jax.experimental import pallas as pl
from jax.experimental.pallas import tpu as pltpu
from jax.experimental.pallas import tpu_sc as plsc

D = 512
SQRT_D = math.sqrt(D)
NC = 2
NS = 16
NW = NC * NS
BB = 8
SLICES = (512, 256, 256)


def _sc_gather_add(tok_table, pos_table, tok_ids, pos_ids, CH):
    n = tok_ids.shape[0]
    per_w = n // NW
    NBUF = 4
    assert per_w * NW == n and per_w % (NBUF * CH) == 0

    mesh = plsc.VectorSubcoreMesh(core_axis_name="c", subcore_axis_name="s")

    rows_t = pltpu.VMEM((CH, D), jnp.float32)
    @functools.partial(
        pl.kernel,
        out_type=jax.ShapeDtypeStruct((n, D), jnp.float32),
        mesh=mesh,
        scratch_types=[
            pltpu.VMEM((per_w,), jnp.int32),
            pltpu.VMEM((per_w,), jnp.int32),
        ] + [rows_t] * (2 * NBUF)
          + [pltpu.SemaphoreType.DMA] * (3 * NBUF),
    )
    def sc_kernel(tok_tbl, pos_tbl, tok_idx, pos_idx, out, ti_all, pi_all,
                  *bufsem):
        trs = bufsem[0:NBUF]
        prs = bufsem[NBUF:2 * NBUF]
        gts = bufsem[2 * NBUF:3 * NBUF]
        gps = bufsem[3 * NBUF:4 * NBUF]
        wss = bufsem[4 * NBUF:5 * NBUF]
        wid = lax.axis_index("s") * NC + lax.axis_index("c")
        base = wid * per_w
        pltpu.sync_copy(tok_idx.at[pl.ds(base, per_w)], ti_all)
        pltpu.sync_copy(pos_idx.at[pl.ds(base, per_w)], pi_all)

        def start_gathers(off, b):
            pltpu.async_copy(tok_tbl.at[ti_all.at[pl.ds(off, CH)]], trs[b],
                             gts[b])
            pltpu.async_copy(pos_tbl.at[pi_all.at[pl.ds(off, CH)]], prs[b],
                             gps[b])

        start_gathers(0, 0)
        start_gathers(CH, 1)

        @pl.loop(0, per_w, step=NBUF * CH)
        def _(c):
            for b in range(NBUF):
                tr, pr = trs[b], prs[b]
                off = c + b * CH
                tb = (b + 2) % NBUF
                nxt = off + 2 * CH

                @pl.when(nxt < per_w)
                def _():
                    @pl.when(off >= 2 * CH)
                    def _():
                        pltpu.make_async_copy(
                            trs[tb],
                            out.at[pl.ds(base + off - 2 * CH, CH)],
                            wss[tb]).wait()
                    start_gathers(nxt, tb)

                pltpu.make_async_copy(
                    tok_tbl.at[ti_all.at[pl.ds(off, CH)]], tr, gts[b]).wait()
                pltpu.make_async_copy(
                    pos_tbl.at[pi_all.at[pl.ds(off, CH)]], pr, gps[b]).wait()

                @pl.loop(0, CH)
                def _(r):
                    trow = tr.at[r]
                    prow = pr.at[r]
                    for cc in range(D // 16):
                        slc = pl.ds(cc * 16, 16)
                        trow[slc] = trow[slc] * SQRT_D + prow[slc]

                pltpu.async_copy(trs[b], out.at[pl.ds(base + off, CH)],
                                 wss[b])

        for b in range(NBUF):
            pltpu.make_async_copy(
                trs[b], out.at[pl.ds(base + per_w - (NBUF - b) * CH, CH)],
                wss[b]).wait()

    return sc_kernel(tok_table, pos_table, tok_ids, pos_ids)


def _tc_body(*refs):
    (part_ref, syn_ref, et_ref, etab_ref, pe_ref, w1_ref, b1_ref, g1_ref,
     bb1_ref, w2_ref, g_ref, bb_ref, out_ref) = refs[-13:]
    s_len = part_ref.shape[1]
    x = part_ref[...]
    etf = et_ref[0].astype(jnp.float32)
    etb = lax.broadcast_in_dim(etf, (BB, s_len, D), (0, 1))
    e0 = etab_ref[0:1, :][None]
    e1 = etab_ref[1:2, :][None]
    e2 = etab_ref[2:3, :][None]
    elem = jnp.where(etb == 0.0, e0, jnp.where(etb == 1.0, e1, e2))

    s2 = syn_ref[...].reshape(BB * s_len, 3)
    h = jnp.dot(s2, w1_ref[...], preferred_element_type=jnp.float32)
    h = h + b1_ref[...]
    m1 = jnp.mean(h, axis=-1, keepdims=True)
    v1 = jnp.mean(h * h, axis=-1, keepdims=True) - m1 * m1
    h = (h - m1) * lax.rsqrt(v1 + 1e-5) * g1_ref[...] + bb1_ref[...]
    h = jnp.maximum(h, 0.0)
    se = jnp.dot(h.astype(jnp.bfloat16), w2_ref[...],
                 preferred_element_type=jnp.float32)
    se = se.reshape(BB, s_len, D)

    comb = x + elem + se + pe_ref[...][None]
    m2 = jnp.mean(comb, axis=-1, keepdims=True)
    v2 = jnp.mean(comb * comb, axis=-1, keepdims=True) - m2 * m2
    out_ref[...] = (comb - m2) * lax.rsqrt(v2 + 1e-5) * g_ref[...] + bb_ref[...]


def _tc_combine_slice(prev, partial, syntax, et3, etab8, pe_s, w1, b1, ln1_g,
                      ln1_b, w2, ln_g, ln_b, blk_off, out_sds):
    sb, s_len, _ = partial.shape
    nb = sb // BB
    grid = (nb,)
    prev_specs = [] if prev is None else [pl.BlockSpec(memory_space=pl.ANY)]
    prev_args = () if prev is None else (prev,)
    aliases = {} if prev is None else {0: 0}
    return pl.pallas_call(
        _tc_body,
        grid=grid,
        in_specs=prev_specs + [
            pl.BlockSpec((BB, s_len, D), lambda i: (i, 0, 0)),
            pl.BlockSpec((BB, s_len, 3), lambda i: (i, 0, 0)),
            pl.BlockSpec((1, BB, s_len), lambda i: (i, 0, 0)),
            pl.BlockSpec((8, D), lambda i: (0, 0)),
            pl.BlockSpec((s_len, D), lambda i: (0, 0)),
            pl.BlockSpec((3, D // 4), lambda i: (0, 0)),
            pl.BlockSpec((1, D // 4), lambda i: (0, 0)),
            pl.BlockSpec((1, D // 4), lambda i: (0, 0)),
            pl.BlockSpec((1, D // 4), lambda i: (0, 0)),
            pl.BlockSpec((D // 4, D), lambda i: (0, 0)),
            pl.BlockSpec((1, D), lambda i: (0, 0)),
            pl.BlockSpec((1, D), lambda i: (0, 0)),
        ],
        out_specs=pl.BlockSpec((BB, s_len, D),
                               lambda i: (i + blk_off, 0, 0)),
        out_shape=out_sds,
        input_output_aliases=aliases,
    )(*prev_args, partial, syntax, et3, etab8, pe_s, w1, b1, ln1_g, ln1_b,
      w2, ln_g, ln_b)


def kernel(token_ids, pos_tags, elem_types, syntax_features, tok_table,
           pos_tag_table, elem_table, pe, w1, b1, ln1_g, ln1_b, w2, b2,
           ln_g, ln_b):
    b_sz, s_len = token_ids.shape
    assert sum(SLICES) == b_sz
    tok_flat = token_ids.reshape(-1).astype(jnp.int32)
    pos_flat = pos_tags.reshape(-1).astype(jnp.int32)
    et_all = elem_types.astype(jnp.int32)
    pe_s = pe[0, :s_len, :] + b2[None, :]
    etab8 = jnp.zeros((8, D), jnp.float32).at[:3].set(elem_table)
    w2b = w2.astype(jnp.bfloat16)
    b1r = b1.reshape(1, -1)
    g1r = ln1_g.reshape(1, -1)
    bb1r = ln1_b.reshape(1, -1)
    gr = ln_g.reshape(1, -1)
    bbr = ln_b.reshape(1, -1)
    out_sds = jax.ShapeDtypeStruct((b_sz, s_len, D), jnp.float32)

    starts = [sum(SLICES[:i]) for i in range(len(SLICES))]
    partials = []
    for st, sb in zip(starts, SLICES):
        n_sl = sb * s_len
        per_w = n_sl // NW
        ch = 16 if per_w % 64 == 0 else 8
        p = _sc_gather_add(tok_table, pos_tag_table,
                           lax.dynamic_slice(tok_flat, (st * s_len,), (n_sl,)),
                           lax.dynamic_slice(pos_flat, (st * s_len,), (n_sl,)),
                           ch)
        partials.append(p.reshape(sb, s_len, D))

    out = None
    for i, (st, sb) in enumerate(zip(starts, SLICES)):
        syn_i = lax.dynamic_slice(syntax_features, (st, 0, 0), (sb, s_len, 3))
        et_i = lax.dynamic_slice(et_all, (st, 0), (sb, s_len))
        et3_i = et_i.reshape(sb // BB, BB, s_len)
        out = _tc_combine_slice(out, partials[i], syn_i, et3_i, etab8, pe_s,
                                w1, b1r, g1r, bb1r, w2b, gr, bbr,
                                blk_off=st // BB, out_sds=out_sds)
    return out
